# Initial kernel scaffold; baseline (speedup 1.0000x reference)
import functools

import jax
import jax.numpy as jnp
from jax import lax
from jax.experimental import pallas as pl
from jax.experimental.pallas import tpu as pltpu
from jax.experimental.pallas import tpu_sc as plsc

NU = 20000
NI = 30000
N = NU + NI
D = 64
H = 32
K = 10
KP = 16
E = 800000
LAYERS = 3

NC, NS = 2, 16
B = 128
CH = 56
NBT = 392
NCH = NBT // CH
NB = NBT * NS
EP = NB * B
TWO_N = 2 * N
RPT = N // NS
FC = 625
NFC = RPT // FC


def _fusion_body(x_ref, a_ref, out_ref):
    x = x_ref[...]
    xsq = jnp.sum(x * x, axis=1, keepdims=True)
    colid = lax.broadcasted_iota(jnp.int32, (NU, KP), 1)
    valid = colid < K
    rowmask = (lax.broadcasted_iota(jnp.int32, (KP, 1), 0) < K).astype(x.dtype)
    cent = x[:KP] * rowmask
    onehot = jnp.zeros((NU, KP), jnp.float32)
    for _ in range(10):
        xc = lax.dot_general(x, cent, (((1,), (1,)), ((), ())),
                             preferred_element_type=jnp.float32)
        csq = jnp.sum(cent * cent, axis=1)[None, :]
        dist = xsq - 2.0 * xc + csq
        dist = jnp.where(valid, dist, jnp.inf)
        assign = jnp.argmin(dist, axis=1)
        onehot = (assign[:, None] == colid).astype(jnp.float32)
        sums = lax.dot_general(onehot, x, (((0,), (0,)), ((), ())),
                               preferred_element_type=jnp.float32)
        counts = jnp.sum(onehot, axis=0)
        cent = sums / jnp.maximum(counts, 1.0)[:, None]
    w = a_ref[...]
    wsums = lax.dot_general(onehot, w * x, (((0,), (0,)), ((), ())),
                            preferred_element_type=jnp.float32)
    wcnt = lax.dot_general(onehot, w, (((0,), (0,)), ((), ())),
                           preferred_element_type=jnp.float32)
    means = jnp.where(wcnt > 0.0, wsums / wcnt, 0.0)
    gath = lax.dot_general(onehot, means, (((1,), (0,)), ((), ())),
                           preferred_element_type=jnp.float32)
    out_ref[...] = w * x + (1.0 - w) * gath


def _fusion_tc(user_emb, alphas):
    return pl.pallas_call(
        _fusion_body,
        out_shape=jax.ShapeDtypeStruct((NU, D), jnp.float32),
    )(user_emb, alphas.reshape(NU, 1))


def _sc_body(x0, colall, rowg, valsg, X, fin, acc,
             colv, rowv, valv, ra, fa, fb, gsem):
    c = lax.axis_index("c")
    s = lax.axis_index("s")

    def _zb(i, _):
        fb[i, 0:16] = jnp.zeros((16,), jnp.float32)
        fb[i, 16:32] = jnp.zeros((16,), jnp.float32)
        return 0
    lax.fori_loop(0, FC, _zb, 0)

    base = c * N + s * RPT
    for k in range(NFC):
        pltpu.sync_copy(x0.at[pl.ds(base + k * FC, FC)], fa)
        pltpu.sync_copy(fa, X.at[pl.ds(base + k * FC, FC)])
        pltpu.sync_copy(fb, acc.at[pl.ds(s * RPT + k * FC, FC)])
    plsc.subcore_barrier()

    for l in range(LAYERS):
        def chunk_body(ch, _):
            b0 = s * NBT + ch * CH
            pltpu.sync_copy(colall.at[l, c, pl.ds(b0, CH)], colv)
            pltpu.sync_copy(rowg.at[pl.ds(b0, CH)], rowv)
            pltpu.sync_copy(valsg.at[pl.ds(b0, CH)], valv)

            def batch_body(j, _):
                pltpu.async_copy(X.at[colv.at[j]], ra, gsem).wait()

                def e_body(e, _):
                    v = valv[j, e]
                    ra[e, 0:16] = ra[e, 0:16] * v
                    ra[e, 16:32] = ra[e, 16:32] * v
                    return 0
                lax.fori_loop(0, B, e_body, 0)
                pltpu.sync_copy(ra, acc.at[rowv.at[j]], add=True)
                return 0
            lax.fori_loop(0, CH, batch_body, 0)
            return 0
        lax.fori_loop(0, NCH, chunk_body, 0)
        plsc.subcore_barrier()

        dst = (l + 1) * TWO_N + c * N + s * RPT
        pltpu.sync_copy(acc.at[pl.ds(s * RPT, RPT)], X.at[pl.ds(dst, RPT)])
        if l != LAYERS - 1:
            for k in range(NFC):
                pltpu.sync_copy(fb, acc.at[pl.ds(s * RPT + k * FC, FC)])
        plsc.subcore_barrier()

    for k in range(NFC):
        r0 = base + k * FC
        pltpu.sync_copy(X.at[pl.ds(r0, FC)], fa)
        for l in range(1, LAYERS + 1):
            pltpu.sync_copy(X.at[pl.ds(l * TWO_N + r0, FC)], fb)

            def _acc(i, _):
                fa[i, 0:16] = fa[i, 0:16] + fb[i, 0:16]
                fa[i, 16:32] = fa[i, 16:32] + fb[i, 16:32]
                return 0
            lax.fori_loop(0, FC, _acc, 0)

        def _scale(i, _):
            fa[i, 0:16] = fa[i, 0:16] * 0.25
            fa[i, 16:32] = fa[i, 16:32] * 0.25
            return 0
        lax.fori_loop(0, FC, _scale, 0)
        pltpu.sync_copy(fa, fin.at[pl.ds(r0, FC)])


_sc_spmm = functools.partial(
    pl.kernel,
    out_type=(jax.ShapeDtypeStruct((4 * TWO_N, H), jnp.float32),
              jax.ShapeDtypeStruct((TWO_N, H), jnp.float32)),
    mesh=plsc.VectorSubcoreMesh(core_axis_name="c", subcore_axis_name="s"),
    scratch_types=[
        pltpu.VMEM_SHARED((N, H), jnp.float32),
        pltpu.VMEM((CH, B), jnp.int32),
        pltpu.VMEM((CH, B), jnp.int32),
        pltpu.VMEM((CH, B), jnp.float32),
        pltpu.VMEM((B, H), jnp.float32),
        pltpu.VMEM((FC, H), jnp.float32),
        pltpu.VMEM((FC, H), jnp.float32),
        pltpu.SemaphoreType.DMA,
    ],
)(_sc_body)


def kernel(adj_indices, adj_values, user_emb, item_emb, alphas):
    fused = _fusion_tc(user_emb, alphas)
    x0 = jnp.concatenate(
        [fused[:, :H], item_emb[:, :H], fused[:, H:], item_emb[:, H:]], axis=0)

    row = adj_indices[0].astype(jnp.int32)
    col = adj_indices[1].astype(jnp.int32)
    pad = EP - E
    colp = jnp.concatenate([col, jnp.zeros((pad,), jnp.int32)])
    rowp = jnp.concatenate([row, jnp.zeros((pad,), jnp.int32)])
    valsp = jnp.concatenate([adj_values, jnp.zeros((pad,), adj_values.dtype)])
    offs = (jnp.arange(LAYERS, dtype=jnp.int32)[:, None] * TWO_N
            + jnp.arange(NC, dtype=jnp.int32)[None, :] * N)
    colall = (colp[None, None, :] + offs[:, :, None]).reshape(LAYERS, NC, NB, B)
    rowg = rowp.reshape(NB, B)
    valsg = valsp.reshape(NB, B)

    _, fin = _sc_spmm(x0, colall, rowg, valsg)
    final = jnp.concatenate([fin[:N], fin[N:]], axis=1)
    return final[:NU], final[NU:]


# speedup vs baseline: 3.9865x; 3.9865x over previous
import jax
import jax.numpy as jnp
from jax import lax
from jax.experimental import pallas as pl
from jax.experimental.pallas import tpu as pltpu
from jax.experimental.pallas import tpu_sc as plsc

NU = 20000
NI = 30000
N = NU + NI
D = 64
H = 32
K = 10
KP = 128
E = 800000
LAYERS = 3

NC, NS = 2, 16
B = 128
CH = 28
NBT = 392
NCH = NBT // CH
NB = NBT * NS
EP = NB * B
NP = 50048
TWO_NP = 2 * NP
RPT = NP // NS
FC = 136
NFC = RPT // FC

BR = 2000
RB = NU // BR


def _fusion_body(x_ref, a_ref, asn_ref, out_ref, sums, counts, means):
    p = pl.program_id(0)
    rb = pl.program_id(1)
    x = x_ref[...]
    w = a_ref[...]
    asn = asn_ref[...]
    colid = lax.broadcasted_iota(jnp.int32, (BR, KP), 1)
    onehot = (asn == colid).astype(jnp.float32)

    @pl.when((p == 0) & (rb == 0))
    def _zero():
        sums[...] = jnp.zeros_like(sums)
        counts[...] = jnp.zeros_like(counts)

    @pl.when(p == 0)
    def _accumulate():
        sums[...] += lax.dot_general(onehot, w * x,
                                     (((0,), (0,)), ((), ())),
                                     preferred_element_type=jnp.float32,
                                     precision=lax.Precision.HIGHEST)
        counts[...] += lax.dot_general(onehot, w,
                                       (((0,), (0,)), ((), ())),
                                       preferred_element_type=jnp.float32,
                                       precision=lax.Precision.HIGHEST)

    @pl.when(p == 1)
    def _emit():
        @pl.when(rb == 0)
        def _means():
            c = counts[...]
            means[...] = jnp.where(c > 0.0, sums[...] / c, 0.0)

        gath = lax.dot_general(onehot, means[...],
                               (((1,), (0,)), ((), ())),
                               preferred_element_type=jnp.float32,
                               precision=lax.Precision.HIGHEST)
        out_ref[...] = w * x + (1.0 - w) * gath


def _fusion_tc(user_emb, alphas, assign):
    return pl.pallas_call(
        _fusion_body,
        grid=(2, RB),
        in_specs=[pl.BlockSpec((BR, D), lambda p, rb: (rb, 0)),
                  pl.BlockSpec((BR, 1), lambda p, rb: (rb, 0)),
                  pl.BlockSpec((BR, 1), lambda p, rb: (rb, 0))],
        out_specs=pl.BlockSpec((BR, D), lambda p, rb: (rb, 0)),
        scratch_shapes=[pltpu.VMEM((KP, D), jnp.float32),
                        pltpu.VMEM((KP, 1), jnp.float32),
                        pltpu.VMEM((KP, D), jnp.float32)],
        out_shape=jax.ShapeDtypeStruct((NU, D), jnp.float32),
    )(user_emb, alphas.reshape(NU, 1), assign.reshape(NU, 1))


def _sc_body(x0, colall, rowg, valsg, X, fin, acc,
             colv, rowv, valv, ra, fa, fb, gsem):
    c = lax.axis_index("c")
    s = lax.axis_index("s")

    def _zb(i, _):
        fb[i, 0:16] = jnp.zeros((16,), jnp.float32)
        fb[i, 16:32] = jnp.zeros((16,), jnp.float32)
        return 0
    lax.fori_loop(0, FC, _zb, 0)

    base = c * NP + s * RPT
    for k in range(NFC):
        pltpu.sync_copy(x0.at[pl.ds(base + k * FC, FC)], fa)
        pltpu.sync_copy(fa, X.at[pl.ds(base + k * FC, FC)])
        pltpu.sync_copy(fb, acc.at[pl.ds(s * RPT + k * FC, FC)])
    plsc.subcore_barrier()

    for l in range(LAYERS):
        def chunk_body(ch, _):
            b0 = s * NBT + ch * CH
            pltpu.sync_copy(colall.at[l, c, pl.ds(b0, CH)], colv)
            pltpu.sync_copy(rowg.at[pl.ds(b0, CH)], rowv)
            pltpu.sync_copy(valsg.at[pl.ds(b0, CH)], valv)

            def batch_body(j, _):
                pltpu.async_copy(X.at[colv.at[j]], ra, gsem).wait()

                def g_body(g, _):
                    vvec = valv[j, pl.ds(g * 16, 16)]
                    for k in range(16):
                        e = g * 16 + k
                        s = vvec[k]
                        ra[e, 0:16] = ra[e, 0:16] * s
                        ra[e, 16:32] = ra[e, 16:32] * s
                    return 0
                lax.fori_loop(0, B // 16, g_body, 0)
                pltpu.sync_copy(ra, acc.at[rowv.at[j]], add=True)
                return 0
            lax.fori_loop(0, CH, batch_body, 0)
            return 0
        lax.fori_loop(0, NCH, chunk_body, 0)
        plsc.subcore_barrier()

        dst = (l + 1) * TWO_NP + c * NP + s * RPT
        pltpu.sync_copy(acc.at[pl.ds(s * RPT, RPT)], X.at[pl.ds(dst, RPT)])
        if l != LAYERS - 1:
            for k in range(NFC):
                pltpu.sync_copy(fb, acc.at[pl.ds(s * RPT + k * FC, FC)])
        plsc.subcore_barrier()

    for k in range(NFC):
        r0 = base + k * FC
        pltpu.sync_copy(X.at[pl.ds(r0, FC)], fa)
        for l in range(1, LAYERS + 1):
            pltpu.sync_copy(X.at[pl.ds(l * TWO_NP + r0, FC)], fb)

            def _acc(i, _):
                fa[i, 0:16] = fa[i, 0:16] + fb[i, 0:16]
                fa[i, 16:32] = fa[i, 16:32] + fb[i, 16:32]
                return 0
            lax.fori_loop(0, FC, _acc, 0)

        def _scale(i, _):
            fa[i, 0:16] = fa[i, 0:16] * 0.25
            fa[i, 16:32] = fa[i, 16:32] * 0.25
            return 0
        lax.fori_loop(0, FC, _scale, 0)
        pltpu.sync_copy(fa, fin.at[pl.ds(r0, FC)])


def _sc_spmm(x0, colall, rowg, valsg):
    return pl.kernel(
        _sc_body,
        out_type=(jax.ShapeDtypeStruct((4 * TWO_NP, H), jnp.float32),
                  jax.ShapeDtypeStruct((TWO_NP, H), jnp.float32)),
        mesh=plsc.VectorSubcoreMesh(core_axis_name="c", subcore_axis_name="s"),
        compiler_params=pltpu.CompilerParams(use_tc_tiling_on_sc=False),
        scratch_types=[
            pltpu.VMEM_SHARED((NP, H), jnp.float32),
            pltpu.VMEM((CH, B), jnp.int32),
            pltpu.VMEM((CH, B), jnp.int32),
            pltpu.VMEM((CH, B), jnp.float32),
            pltpu.VMEM((B, H), jnp.float32),
            pltpu.VMEM((FC, H), jnp.float32),
            pltpu.VMEM((FC, H), jnp.float32),
            pltpu.SemaphoreType.DMA,
        ],
    )(x0, colall, rowg, valsg)


def _kmeans_assign_trajectory(x):
    cent = x[:K]
    assign = jnp.zeros((x.shape[0],), dtype=jnp.int32)
    for _ in range(10):
        d = (jnp.sum(x * x, axis=1, keepdims=True)
             - 2.0 * (x @ cent.T)
             + jnp.sum(cent * cent, axis=1)[None, :])
        assign = jnp.argmin(d, axis=1)
        sums = jax.ops.segment_sum(x, assign, num_segments=K)
        counts = jax.ops.segment_sum(jnp.ones((x.shape[0],), x.dtype), assign,
                                     num_segments=K)
        cent = sums / jnp.maximum(counts, 1.0)[:, None]
    return assign.astype(jnp.int32)


def kernel(adj_indices, adj_values, user_emb, item_emb, alphas):
    assign = _kmeans_assign_trajectory(user_emb)
    fused = _fusion_tc(user_emb, alphas, assign)

    zpad = jnp.zeros((NP - N, H), jnp.float32)
    x0 = jnp.concatenate(
        [fused[:, :H], item_emb[:, :H], zpad,
         fused[:, H:], item_emb[:, H:], zpad], axis=0)

    row = adj_indices[0].astype(jnp.int32)
    col = adj_indices[1].astype(jnp.int32)
    pad = EP - E
    colp = jnp.concatenate([col, jnp.zeros((pad,), jnp.int32)])
    rowp = jnp.concatenate([row, jnp.zeros((pad,), jnp.int32)])
    valsp = jnp.concatenate([adj_values, jnp.zeros((pad,), adj_values.dtype)])
    offs = (jnp.arange(LAYERS, dtype=jnp.int32)[:, None] * TWO_NP
            + jnp.arange(NC, dtype=jnp.int32)[None, :] * NP)
    colall = (colp[None, None, :] + offs[:, :, None]).reshape(LAYERS, NC, NB, B)
    rowg = rowp.reshape(NB, B)
    valsg = valsp.reshape(NB, B)

    _, fin = _sc_spmm(x0, colall, rowg, valsg)
    final = jnp.concatenate([fin[:N], fin[NP:NP + N]], axis=1)
    return final[:NU], final[NU:]
